# baseline (speedup 1.0000x reference)
import functools

import jax
import jax.numpy as jnp
from jax import lax
from jax.experimental import pallas as pl
from jax.experimental.pallas import tpu as pltpu
from jax.experimental.pallas import tpu_sc as plsc

B = 16384
C = 1000
NBINS = 10
MINCLAMP = 0.01
MAXCLAMP = 100.0

NC = 2
NS = 16
L = 16
NW = NC * NS
RW = B // NW
RC = 32
NCHUNK = RW // RC

OFFS = tuple(16 * j for j in range(C // L)) + (C - L,)

_MESH = dict(core_axis_name="c", subcore_axis_name="s", num_cores=NC,
             num_subcores=NS)


def _row_max_chunk(buf, accs2d, maxes_v, local_base, accmin, accmax):
    def row_body(r, carry):
        chains = [None, None, None, None]
        for k, off in enumerate(OFFS):
            v = buf[r, pl.ds(off, L)]
            c = k & 3
            chains[c] = v if chains[c] is None else jnp.maximum(chains[c], v)
        acc = jnp.maximum(jnp.maximum(chains[0], chains[1]),
                          jnp.maximum(chains[2], chains[3]))
        accs2d[r, :] = acc
        return carry

    lax.fori_loop(0, RC, row_body, 0, unroll=False)

    iota = lax.iota(jnp.int32, L)
    for g in range(RC // L):
        rows = iota + (g * L)
        m0 = None
        m1 = None
        for col in range(L):
            v = plsc.load_gather(accs2d, [rows, jnp.full((L,), col, jnp.int32)])
            if col & 1 == 0:
                m0 = v if m0 is None else jnp.maximum(m0, v)
            else:
                m1 = v if m1 is None else jnp.maximum(m1, v)
        m = jnp.maximum(m0, m1)
        maxes_v[pl.ds(local_base + g * L, L)] = m
        accmin = jnp.minimum(accmin, m)
        accmax = jnp.maximum(accmax, m)
    return accmin, accmax


@functools.partial(
    pl.kernel,
    out_type=(
        jax.ShapeDtypeStruct((B,), jnp.float32),
        jax.ShapeDtypeStruct((NW, 2, L), jnp.float32),
    ),
    mesh=plsc.VectorSubcoreMesh(**_MESH),
    scratch_types=[
        pltpu.VMEM((RC, C), jnp.float32),
        pltpu.VMEM((RC, C), jnp.float32),
        pltpu.VMEM((RC, L), jnp.float32),
        pltpu.VMEM((RW,), jnp.float32),
        pltpu.VMEM((2, L), jnp.float32),
        pltpu.SemaphoreType.DMA,
        pltpu.SemaphoreType.DMA,
    ],
    compiler_params=pltpu.CompilerParams(needs_layout_passes=False,
                                         use_tc_tiling_on_sc=True),
)
def _rowmax_kernel(probs_hbm, maxes_hbm, part_hbm,
                   buf0, buf1, accs2d, maxes_v, pbuf, sem0, sem1):
    wid = lax.axis_index("c") * NS + lax.axis_index("s")
    rbase = wid * RW

    def start(c, buf, sem):
        pltpu.make_async_copy(
            probs_hbm.at[pl.ds(rbase + c * RC, RC), :], buf, sem).start()

    def wait(buf, sem):
        pltpu.make_async_copy(
            probs_hbm.at[pl.ds(rbase, RC), :], buf, sem).wait()

    start(0, buf0, sem0)

    inf = jnp.full((L,), jnp.inf, jnp.float32)

    def round_body(i, carry):
        accmin, accmax = carry
        c0 = 2 * i
        c1 = c0 + 1
        start(c1, buf1, sem1)
        wait(buf0, sem0)
        accmin, accmax = _row_max_chunk(buf0, accs2d, maxes_v, c0 * RC,
                                        accmin, accmax)

        @pl.when(i < NCHUNK // 2 - 1)
        def _():
            start(c1 + 1, buf0, sem0)

        wait(buf1, sem1)
        accmin, accmax = _row_max_chunk(buf1, accs2d, maxes_v, c1 * RC,
                                        accmin, accmax)
        return accmin, accmax

    accmin, accmax = lax.fori_loop(0, NCHUNK // 2, round_body, (inf, -inf))

    pbuf[0, :] = accmin
    pbuf[1, :] = accmax
    pltpu.sync_copy(maxes_v, maxes_hbm.at[pl.ds(rbase, RW)])
    pltpu.sync_copy(pbuf, part_hbm.at[wid])


def _calibrate_tc_body(maxes_ref, part_ref, edges_ref, bp_ref, out_ref):
    pr = part_ref[...]
    gmn = jnp.min(pr[0, :])
    gmx = jnp.max(pr[1, :])
    denom = gmx - gmn + jnp.float32(1e-12)
    x = (maxes_ref[...] - gmn) / denom
    cnt = jnp.zeros(x.shape, jnp.int32)
    for i in range(NBINS + 1):
        e = edges_ref[0, i]
        cnt = cnt + jnp.where(e < x, jnp.int32(1), jnp.int32(0))
    idx = jnp.clip(cnt - 1, 0, NBINS - 1)
    bp = bp_ref[...]
    sc = jnp.float32(1.0) / jnp.clip(bp * bp, jnp.float32(MINCLAMP),
                                     jnp.float32(MAXCLAMP))
    scale = jnp.zeros(x.shape, jnp.float32)
    for b in range(NBINS):
        scale = jnp.where(idx == b, sc[0, b], scale)
    out_ref[...] = x * scale


def _calibrate_tc(maxes2d, part2d, edges2d, bp2d):
    return pl.pallas_call(
        _calibrate_tc_body,
        out_shape=jax.ShapeDtypeStruct((B // 128, 128), jnp.float32),
    )(maxes2d, part2d, edges2d, bp2d)


def kernel(probabilities, bin_params):
    edges = jnp.linspace(0.0, 1.0, NBINS + 1, dtype=jnp.float32)
    ed2d = jnp.zeros((8, 128), jnp.float32).at[0, : NBINS + 1].set(edges)
    bp2d = jnp.zeros((8, 128), jnp.float32).at[0, :NBINS].set(bin_params)
    maxes, partials = _rowmax_kernel(probabilities)
    part2d = partials.transpose(1, 0, 2).reshape(2, NW * L)
    out = _calibrate_tc(maxes.reshape(B // 128, 128), part2d, ed2d,
                        bp2d).reshape(B)
    calibrated = jnp.zeros_like(probabilities)
    return (out, calibrated)


# speedup vs baseline: 1.0393x; 1.0393x over previous
import functools

import jax
import jax.numpy as jnp
from jax import lax
from jax.experimental import pallas as pl
from jax.experimental.pallas import tpu as pltpu
from jax.experimental.pallas import tpu_sc as plsc

B = 16384
C = 1000
NBINS = 10
MINCLAMP = 0.01
MAXCLAMP = 100.0

NC = 2
NS = 16
L = 16
NW = NC * NS
RW = B // NW

RBLK = 1024


def _rowmax_tc_body(p_ref, out_ref):
    x = p_ref[...]
    out_ref[...] = jnp.max(x, axis=1).reshape(RBLK // 128, 128)


def _rowmax_tc(probs):
    return pl.pallas_call(
        _rowmax_tc_body,
        grid=(B // RBLK,),
        in_specs=[pl.BlockSpec((RBLK, C), lambda g: (g, 0))],
        out_specs=pl.BlockSpec((RBLK // 128, 128), lambda g: (g, 0)),
        out_shape=jax.ShapeDtypeStruct((B // 128, 128), jnp.float32),
    )(probs)


@functools.partial(
    pl.kernel,
    out_type=jax.ShapeDtypeStruct((B,), jnp.float32),
    mesh=plsc.VectorSubcoreMesh(core_axis_name="c", subcore_axis_name="s",
                                num_cores=NC, num_subcores=NS),
    scratch_types=[
        pltpu.VMEM((B,), jnp.float32),
        pltpu.VMEM((L,), jnp.float32),
        pltpu.VMEM((L,), jnp.float32),
        pltpu.VMEM((L,), jnp.float32),
        pltpu.VMEM((RW,), jnp.float32),
    ],
    compiler_params=pltpu.CompilerParams(needs_layout_passes=False),
)
def _calibrate_sc(maxes_hbm, edges_hbm, bp_hbm, out_hbm,
                  m_v, ed_v, bp_v, sc_v, out_v):
    wid = lax.axis_index("c") * NS + lax.axis_index("s")
    rbase = wid * RW

    pltpu.sync_copy(maxes_hbm, m_v)
    pltpu.sync_copy(edges_hbm, ed_v)
    pltpu.sync_copy(bp_hbm, bp_v)

    def red_body(k, carry):
        mn0, mn1, mx0, mx1 = carry
        v0 = m_v[pl.ds(k * 4 * L, L)]
        v1 = m_v[pl.ds(k * 4 * L + L, L)]
        v2 = m_v[pl.ds(k * 4 * L + 2 * L, L)]
        v3 = m_v[pl.ds(k * 4 * L + 3 * L, L)]
        return (jnp.minimum(jnp.minimum(mn0, v0), v2),
                jnp.minimum(jnp.minimum(mn1, v1), v3),
                jnp.maximum(jnp.maximum(mx0, v0), v2),
                jnp.maximum(jnp.maximum(mx1, v1), v3))

    inf = jnp.full((L,), jnp.inf, jnp.float32)
    mn0, mn1, mx0, mx1 = lax.fori_loop(0, B // (4 * L), red_body,
                                       (inf, inf, -inf, -inf))
    gmn = jnp.min(jnp.minimum(mn0, mn1))
    gmx = jnp.max(jnp.maximum(mx0, mx1))
    denom = gmx - gmn + jnp.float32(1e-12)

    bp = bp_v[:]
    sc_v[:] = jnp.float32(1.0) / jnp.clip(bp * bp, jnp.float32(MINCLAMP),
                                          jnp.float32(MAXCLAMP))
    ed = ed_v[:]
    edges = [ed[i] for i in range(NBINS + 1)]

    def vec_body(k, carry):
        x = (m_v[pl.ds(rbase + k * L, L)] - gmn) / denom
        cnt = jnp.zeros((L,), jnp.int32)
        for e in edges:
            cnt = cnt + jnp.where(e < x, jnp.int32(1), jnp.int32(0))
        idx = jnp.clip(cnt - 1, 0, NBINS - 1)
        g = plsc.load_gather(sc_v, [idx])
        out_v[pl.ds(k * L, L)] = x * g
        return carry

    lax.fori_loop(0, RW // L, vec_body, 0)
    pltpu.sync_copy(out_v, out_hbm.at[pl.ds(rbase, RW)])


def kernel(probabilities, bin_params):
    edges = jnp.linspace(0.0, 1.0, NBINS + 1, dtype=jnp.float32)
    ed16 = jnp.zeros((L,), jnp.float32).at[: NBINS + 1].set(edges)
    bp16 = jnp.zeros((L,), jnp.float32).at[:NBINS].set(bin_params)
    maxes = _rowmax_tc(probabilities).reshape(B)
    out = _calibrate_sc(maxes, ed16, bp16)
    calibrated = jnp.zeros_like(probabilities)
    return (out, calibrated)


# speedup vs baseline: 1.9362x; 1.8630x over previous
import functools

import jax
import jax.numpy as jnp
from jax import lax
from jax.experimental import pallas as pl
from jax.experimental.pallas import tpu as pltpu
from jax.experimental.pallas import tpu_sc as plsc

B = 16384
C = 1000
NBINS = 10
MINCLAMP = 0.01
MAXCLAMP = 100.0

NC = 2
NS = 16
L = 16
NW = NC * NS
RW = B // NW

CBLK = 2048


def _rowmax_tc_body(p_ref, out_ref):
    x = p_ref[...]
    out_ref[...] = jnp.max(x, axis=0).reshape(CBLK // 128, 128)


def _rowmax_tc(probs_t):
    return pl.pallas_call(
        _rowmax_tc_body,
        grid=(B // CBLK,),
        in_specs=[pl.BlockSpec((C, CBLK), lambda g: (0, g))],
        out_specs=pl.BlockSpec((CBLK // 128, 128), lambda g: (g, 0)),
        out_shape=jax.ShapeDtypeStruct((B // 128, 128), jnp.float32),
    )(probs_t)


@functools.partial(
    pl.kernel,
    out_type=jax.ShapeDtypeStruct((B,), jnp.float32),
    mesh=plsc.VectorSubcoreMesh(core_axis_name="c", subcore_axis_name="s",
                                num_cores=NC, num_subcores=NS),
    scratch_types=[
        pltpu.VMEM((B,), jnp.float32),
        pltpu.VMEM((L,), jnp.float32),
        pltpu.VMEM((L,), jnp.float32),
        pltpu.VMEM((L,), jnp.float32),
        pltpu.VMEM((RW,), jnp.float32),
    ],
    compiler_params=pltpu.CompilerParams(needs_layout_passes=False),
)
def _calibrate_sc(maxes_hbm, edges_hbm, bp_hbm, out_hbm,
                  m_v, ed_v, bp_v, sc_v, out_v):
    wid = lax.axis_index("c") * NS + lax.axis_index("s")
    rbase = wid * RW

    pltpu.sync_copy(maxes_hbm, m_v)
    pltpu.sync_copy(edges_hbm, ed_v)
    pltpu.sync_copy(bp_hbm, bp_v)

    def red_body(k, carry):
        mn0, mn1, mx0, mx1 = carry
        v0 = m_v[pl.ds(k * 4 * L, L)]
        v1 = m_v[pl.ds(k * 4 * L + L, L)]
        v2 = m_v[pl.ds(k * 4 * L + 2 * L, L)]
        v3 = m_v[pl.ds(k * 4 * L + 3 * L, L)]
        return (jnp.minimum(jnp.minimum(mn0, v0), v2),
                jnp.minimum(jnp.minimum(mn1, v1), v3),
                jnp.maximum(jnp.maximum(mx0, v0), v2),
                jnp.maximum(jnp.maximum(mx1, v1), v3))

    inf = jnp.full((L,), jnp.inf, jnp.float32)
    mn0, mn1, mx0, mx1 = lax.fori_loop(0, B // (4 * L), red_body,
                                       (inf, inf, -inf, -inf))
    gmn = jnp.min(jnp.minimum(mn0, mn1))
    gmx = jnp.max(jnp.maximum(mx0, mx1))
    denom = gmx - gmn + jnp.float32(1e-12)

    bp = bp_v[:]
    sc_v[:] = jnp.float32(1.0) / jnp.clip(bp * bp, jnp.float32(MINCLAMP),
                                          jnp.float32(MAXCLAMP))
    ed = ed_v[:]
    edges = [ed[i] for i in range(NBINS + 1)]

    def vec_body(k, carry):
        x = (m_v[pl.ds(rbase + k * L, L)] - gmn) / denom
        cnt = jnp.zeros((L,), jnp.int32)
        for e in edges:
            cnt = cnt + jnp.where(e < x, jnp.int32(1), jnp.int32(0))
        idx = jnp.clip(cnt - 1, 0, NBINS - 1)
        g = plsc.load_gather(sc_v, [idx])
        out_v[pl.ds(k * L, L)] = x * g
        return carry

    lax.fori_loop(0, RW // L, vec_body, 0)
    pltpu.sync_copy(out_v, out_hbm.at[pl.ds(rbase, RW)])


def kernel(probabilities, bin_params):
    edges = jnp.linspace(0.0, 1.0, NBINS + 1, dtype=jnp.float32)
    ed16 = jnp.zeros((L,), jnp.float32).at[: NBINS + 1].set(edges)
    bp16 = jnp.zeros((L,), jnp.float32).at[:NBINS].set(bin_params)
    maxes = _rowmax_tc(probabilities.T).reshape(B)
    out = _calibrate_sc(maxes, ed16, bp16)
    calibrated = jnp.zeros_like(probabilities)
    return (out, calibrated)


# speedup vs baseline: 2.0568x; 1.0623x over previous
import functools

import jax
import jax.numpy as jnp
from jax import lax
from jax.experimental import pallas as pl
from jax.experimental.pallas import tpu as pltpu
from jax.experimental.pallas import tpu_sc as plsc

B = 16384
C = 1000
NBINS = 10
MINCLAMP = 0.01
MAXCLAMP = 100.0

NC = 2
NS = 16
L = 16
NW = NC * NS
RW = B // NW

CBLK = 2048


def _rowmax_tc_body(p_ref, out_ref, mm_ref, accn_ref, accx_ref):
    g = pl.program_id(0)
    x = p_ref[...]
    m = jnp.max(x, axis=0).reshape(CBLK // 128, 128)
    out_ref[...] = m

    @pl.when(g == 0)
    def _():
        accn_ref[...] = m
        accx_ref[...] = m

    @pl.when(g > 0)
    def _():
        accn_ref[...] = jnp.minimum(accn_ref[...], m)
        accx_ref[...] = jnp.maximum(accx_ref[...], m)

    @pl.when(g == pl.num_programs(0) - 1)
    def _():
        gmn = jnp.min(accn_ref[...])
        gmx = jnp.max(accx_ref[...])
        col = lax.broadcasted_iota(jnp.int32, (8, 128), 1)
        mm_ref[...] = jnp.where(col == 1, gmx, gmn)


def _rowmax_tc(probs_t):
    return pl.pallas_call(
        _rowmax_tc_body,
        grid=(B // CBLK,),
        in_specs=[pl.BlockSpec((C, CBLK), lambda g: (0, g))],
        out_specs=[
            pl.BlockSpec((CBLK // 128, 128), lambda g: (g, 0)),
            pl.BlockSpec((8, 128), lambda g: (0, 0)),
        ],
        out_shape=[
            jax.ShapeDtypeStruct((B // 128, 128), jnp.float32),
            jax.ShapeDtypeStruct((8, 128), jnp.float32),
        ],
        scratch_shapes=[
            pltpu.VMEM((CBLK // 128, 128), jnp.float32),
            pltpu.VMEM((CBLK // 128, 128), jnp.float32),
        ],
    )(probs_t)


@functools.partial(
    pl.kernel,
    out_type=jax.ShapeDtypeStruct((B,), jnp.float32),
    mesh=plsc.VectorSubcoreMesh(core_axis_name="c", subcore_axis_name="s",
                                num_cores=NC, num_subcores=NS),
    scratch_types=[
        pltpu.VMEM((RW,), jnp.float32),
        pltpu.VMEM((L,), jnp.float32),
        pltpu.VMEM((L,), jnp.float32),
        pltpu.VMEM((L,), jnp.float32),
        pltpu.VMEM((L,), jnp.float32),
        pltpu.VMEM((RW,), jnp.float32),
    ],
    compiler_params=pltpu.CompilerParams(needs_layout_passes=False),
    cost_estimate=pl.CostEstimate(flops=400_000, bytes_accessed=140_000,
                                  transcendentals=0),
)
def _calibrate_sc(maxes_hbm, mm_hbm, edges_hbm, bp_hbm, out_hbm,
                  m_v, mm_v, ed_v, bp_v, sc_v, out_v):
    wid = lax.axis_index("c") * NS + lax.axis_index("s")
    rbase = wid * RW

    pltpu.sync_copy(maxes_hbm.at[pl.ds(rbase, RW)], m_v)
    pltpu.sync_copy(mm_hbm.at[0, pl.ds(0, L)], mm_v)
    pltpu.sync_copy(edges_hbm, ed_v)
    pltpu.sync_copy(bp_hbm, bp_v)

    mm = mm_v[:]
    gmn = mm[0]
    gmx = mm[1]
    denom = gmx - gmn + jnp.float32(1e-12)

    bp = bp_v[:]
    sc_v[:] = jnp.float32(1.0) / jnp.clip(bp * bp, jnp.float32(MINCLAMP),
                                          jnp.float32(MAXCLAMP))
    ed = ed_v[:]
    edges = [ed[i] for i in range(NBINS + 1)]

    def vec_body(k, carry):
        x = (m_v[pl.ds(k * L, L)] - gmn) / denom
        cnt = jnp.zeros((L,), jnp.int32)
        for e in edges:
            cnt = cnt + jnp.where(e < x, jnp.int32(1), jnp.int32(0))
        idx = jnp.clip(cnt - 1, 0, NBINS - 1)
        g = plsc.load_gather(sc_v, [idx])
        out_v[pl.ds(k * L, L)] = x * g
        return carry

    lax.fori_loop(0, RW // L, vec_body, 0)
    pltpu.sync_copy(out_v, out_hbm.at[pl.ds(rbase, RW)])


def kernel(probabilities, bin_params):
    edges = jnp.linspace(0.0, 1.0, NBINS + 1, dtype=jnp.float32)
    ed16 = jnp.zeros((L,), jnp.float32).at[: NBINS + 1].set(edges)
    bp16 = jnp.zeros((L,), jnp.float32).at[:NBINS].set(bin_params)
    maxes2d, mm2d = _rowmax_tc(probabilities.T)
    out = _calibrate_sc(maxes2d.reshape(B), mm2d, ed16, bp16)
    calibrated = jnp.zeros_like(probabilities)
    return (out, calibrated)


# speedup vs baseline: 2.1080x; 1.0249x over previous
import functools

import jax
import jax.numpy as jnp
from jax import lax
from jax.experimental import pallas as pl
from jax.experimental.pallas import tpu as pltpu
from jax.experimental.pallas import tpu_sc as plsc

B = 16384
C = 1000
NBINS = 10
MINCLAMP = 0.01
MAXCLAMP = 100.0

NC = 2
NS = 16
L = 16
NW = NC * NS
RW = B // NW

CBLK = 2048


def _rowmax_tc_body(p_ref, out_ref, mm_ref, z_ref, accn_ref, accx_ref):
    g = pl.program_id(0)
    x = p_ref[...]
    m = jnp.max(x, axis=0).reshape(CBLK // 128, 128)
    out_ref[...] = m
    z_ref[...] = jnp.zeros((C, CBLK), jnp.float32)

    @pl.when(g == 0)
    def _():
        accn_ref[...] = m
        accx_ref[...] = m

    @pl.when(g > 0)
    def _():
        accn_ref[...] = jnp.minimum(accn_ref[...], m)
        accx_ref[...] = jnp.maximum(accx_ref[...], m)

    @pl.when(g == pl.num_programs(0) - 1)
    def _():
        gmn = jnp.min(accn_ref[...])
        gmx = jnp.max(accx_ref[...])
        col = lax.broadcasted_iota(jnp.int32, (8, 128), 1)
        mm_ref[...] = jnp.where(col == 1, gmx, gmn)


def _rowmax_tc(probs_t):
    return pl.pallas_call(
        _rowmax_tc_body,
        grid=(B // CBLK,),
        in_specs=[pl.BlockSpec((C, CBLK), lambda g: (0, g))],
        out_specs=[
            pl.BlockSpec((CBLK // 128, 128), lambda g: (g, 0)),
            pl.BlockSpec((8, 128), lambda g: (0, 0)),
            pl.BlockSpec((C, CBLK), lambda g: (0, g)),
        ],
        out_shape=[
            jax.ShapeDtypeStruct((B // 128, 128), jnp.float32),
            jax.ShapeDtypeStruct((8, 128), jnp.float32),
            jax.ShapeDtypeStruct((C, B), jnp.float32),
        ],
        scratch_shapes=[
            pltpu.VMEM((CBLK // 128, 128), jnp.float32),
            pltpu.VMEM((CBLK // 128, 128), jnp.float32),
        ],
    )(probs_t)


@functools.partial(
    pl.kernel,
    out_type=jax.ShapeDtypeStruct((B,), jnp.float32),
    mesh=plsc.VectorSubcoreMesh(core_axis_name="c", subcore_axis_name="s",
                                num_cores=NC, num_subcores=NS),
    scratch_types=[
        pltpu.VMEM((RW,), jnp.float32),
        pltpu.VMEM((L,), jnp.float32),
        pltpu.VMEM((L,), jnp.float32),
        pltpu.VMEM((L,), jnp.float32),
        pltpu.VMEM((L,), jnp.float32),
        pltpu.VMEM((RW,), jnp.float32),
    ],
    compiler_params=pltpu.CompilerParams(needs_layout_passes=False),
    cost_estimate=pl.CostEstimate(flops=400_000, bytes_accessed=140_000,
                                  transcendentals=0),
)
def _calibrate_sc(maxes_hbm, mm_hbm, edges_hbm, bp_hbm, out_hbm,
                  m_v, mm_v, ed_v, bp_v, sc_v, out_v):
    wid = lax.axis_index("c") * NS + lax.axis_index("s")
    rbase = wid * RW

    pltpu.sync_copy(maxes_hbm.at[pl.ds(rbase, RW)], m_v)
    pltpu.sync_copy(mm_hbm.at[0, pl.ds(0, L)], mm_v)
    pltpu.sync_copy(edges_hbm, ed_v)
    pltpu.sync_copy(bp_hbm, bp_v)

    mm = mm_v[:]
    gmn = mm[0]
    gmx = mm[1]
    denom = gmx - gmn + jnp.float32(1e-12)

    bp = bp_v[:]
    sc_v[:] = jnp.float32(1.0) / jnp.clip(bp * bp, jnp.float32(MINCLAMP),
                                          jnp.float32(MAXCLAMP))
    ed = ed_v[:]
    edges = [ed[i] for i in range(NBINS + 1)]

    def vec_body(k, carry):
        x = (m_v[pl.ds(k * L, L)] - gmn) / denom
        cnt = jnp.zeros((L,), jnp.int32)
        for e in edges:
            cnt = cnt + jnp.where(e < x, jnp.int32(1), jnp.int32(0))
        idx = jnp.clip(cnt - 1, 0, NBINS - 1)
        g = plsc.load_gather(sc_v, [idx])
        out_v[pl.ds(k * L, L)] = x * g
        return carry

    lax.fori_loop(0, RW // L, vec_body, 0)
    pltpu.sync_copy(out_v, out_hbm.at[pl.ds(rbase, RW)])


def kernel(probabilities, bin_params):
    edges = jnp.linspace(0.0, 1.0, NBINS + 1, dtype=jnp.float32)
    ed16 = jnp.zeros((L,), jnp.float32).at[: NBINS + 1].set(edges)
    bp16 = jnp.zeros((L,), jnp.float32).at[:NBINS].set(bin_params)
    maxes2d, mm2d, zeros_t = _rowmax_tc(probabilities.T)
    out = _calibrate_sc(maxes2d.reshape(B), mm2d, ed16, bp16)
    calibrated = zeros_t.T
    return (out, calibrated)


# speedup vs baseline: 2.2061x; 1.0465x over previous
import functools

import jax
import jax.numpy as jnp
from jax import lax
from jax.experimental import pallas as pl
from jax.experimental.pallas import tpu as pltpu
from jax.experimental.pallas import tpu_sc as plsc

B = 16384
C = 1000
NBINS = 10
MINCLAMP = 0.01
MAXCLAMP = 100.0

NC = 2
NS = 16
L = 16
NW = NC * NS
RW = B // NW

EDGES = (0.0, 0.10000000149011612, 0.20000000298023224, 0.30000001192092896,
         0.4000000059604645, 0.5, 0.6000000238418579, 0.699999988079071,
         0.800000011920929, 0.9000000357627869, 1.0)

CBLK = 2048


def _rowmax_tc_body(p_ref, out_ref, mm_ref, z_ref, accn_ref, accx_ref):
    g = pl.program_id(0)
    x = p_ref[...]
    m = jnp.max(x, axis=0).reshape(CBLK // 128, 128)
    out_ref[...] = m
    z_ref[...] = jnp.zeros((C, CBLK), jnp.float32)

    @pl.when(g == 0)
    def _():
        accn_ref[...] = m
        accx_ref[...] = m

    @pl.when(g > 0)
    def _():
        accn_ref[...] = jnp.minimum(accn_ref[...], m)
        accx_ref[...] = jnp.maximum(accx_ref[...], m)

    @pl.when(g == pl.num_programs(0) - 1)
    def _():
        gmn = jnp.min(accn_ref[...])
        gmx = jnp.max(accx_ref[...])
        col = lax.broadcasted_iota(jnp.int32, (8, 128), 1)
        mm_ref[...] = jnp.where(col == 1, gmx, gmn)


def _rowmax_tc(probs_t):
    return pl.pallas_call(
        _rowmax_tc_body,
        grid=(B // CBLK,),
        in_specs=[pl.BlockSpec((C, CBLK), lambda g: (0, g))],
        out_specs=[
            pl.BlockSpec((CBLK // 128, 128), lambda g: (g, 0)),
            pl.BlockSpec((8, 128), lambda g: (0, 0)),
            pl.BlockSpec((C, CBLK), lambda g: (0, g)),
        ],
        out_shape=[
            jax.ShapeDtypeStruct((B // 128, 128), jnp.float32),
            jax.ShapeDtypeStruct((8, 128), jnp.float32),
            jax.ShapeDtypeStruct((C, B), jnp.float32),
        ],
        scratch_shapes=[
            pltpu.VMEM((CBLK // 128, 128), jnp.float32),
            pltpu.VMEM((CBLK // 128, 128), jnp.float32),
        ],
    )(probs_t)


@functools.partial(
    pl.kernel,
    out_type=jax.ShapeDtypeStruct((B,), jnp.float32),
    mesh=plsc.VectorSubcoreMesh(core_axis_name="c", subcore_axis_name="s",
                                num_cores=NC, num_subcores=NS),
    scratch_types=[
        pltpu.VMEM((RW,), jnp.float32),
        pltpu.VMEM((L,), jnp.float32),
        pltpu.VMEM((L,), jnp.float32),
        pltpu.VMEM((L,), jnp.float32),
        pltpu.VMEM((RW,), jnp.float32),
    ],
    compiler_params=pltpu.CompilerParams(needs_layout_passes=False),
)
def _calibrate_sc(maxes_hbm, mm_hbm, bp_hbm, out_hbm,
                  m_v, mm_v, bp_v, sc_v, out_v):
    wid = lax.axis_index("c") * NS + lax.axis_index("s")
    rbase = wid * RW

    pltpu.sync_copy(maxes_hbm.at[pl.ds(rbase, RW)], m_v)
    pltpu.sync_copy(mm_hbm.at[0, pl.ds(0, L)], mm_v)
    pltpu.sync_copy(bp_hbm, bp_v.at[pl.ds(0, NBINS)])

    mm = mm_v[:]
    gmn = mm[0]
    gmx = mm[1]
    denom_v = jnp.zeros((L,), jnp.float32) + (gmx - gmn + jnp.float32(1e-12))
    inv = jnp.full((L,), 1.0, jnp.float32) / denom_v

    bp = bp_v[:]
    sc_v[:] = jnp.float32(1.0) / jnp.clip(bp * bp, jnp.float32(MINCLAMP),
                                          jnp.float32(MAXCLAMP))

    def vec_body(k, carry):
        x = (m_v[pl.ds(k * L, L)] - gmn) * inv
        cnt = jnp.zeros((L,), jnp.int32)
        for e in EDGES:
            cnt = cnt + jnp.where(jnp.float32(e) < x, jnp.int32(1),
                                  jnp.int32(0))
        idx = jnp.clip(cnt - 1, 0, NBINS - 1)
        g = plsc.load_gather(sc_v, [idx])
        out_v[pl.ds(k * L, L)] = x * g
        return carry

    lax.fori_loop(0, RW // L, vec_body, 0)
    pltpu.sync_copy(out_v, out_hbm.at[pl.ds(rbase, RW)])


def kernel(probabilities, bin_params):
    maxes2d, mm2d, zeros_t = _rowmax_tc(probabilities.T)
    out = _calibrate_sc(maxes2d.reshape(B), mm2d, bin_params)
    calibrated = zeros_t.T
    return (out, calibrated)
